# Initial kernel scaffold; baseline (speedup 1.0000x reference)
import jax
import jax.numpy as jnp
from jax.experimental import pallas as pl


def kernel(inputs, table_activity, table_resource):
    raise NotImplementedError("write your pallas kernel here")


# speedup vs baseline: 1.2021x; 1.2021x over previous
import functools

import jax
import jax.numpy as jnp
import numpy as np
from jax import lax
from jax.experimental import pallas as pl
from jax.experimental.pallas import tpu as pltpu
from jax.experimental.pallas import tpu_sc as plsc

B, C, A, D = 1024, 50, 4, 64
VOCAB0, VOCAB1 = 100000, 1000
CA = C * A

NC, NS = 2, 16
NW = NC * NS
BPW = B // NW
NB = 4
NCHUNK = BPW // NB
EV = NB * C
EV_PAD = 224
HALF = 112


def _pos_encoding_np():
    pos = np.arange(C)[:, np.newaxis].astype(np.float32)
    i = np.arange(D)[np.newaxis, :].astype(np.float32)
    angle = pos / np.power(10000, 2.0 * (np.floor(i / 2.0)) / np.float32(D))
    angle[:, 0::2] = np.sin(angle[:, 0::2])
    angle[:, 1::2] = np.cos(angle[:, 1::2])
    return angle


_POS = _pos_encoding_np()

_E = np.arange(EV_PAD)
_WMAP = (np.minimum(_E // C, NB - 1) * CA + (_E % C) * A).astype(np.int32)


def _sc_body(inp_hbm, ta_hbm, tr_hbm, pos_hbm, wmap_hbm, out_hbm,
             inp_v, idx0_v, idx1_v, st0_v, st1_v, buf_v, pos_v, wmap_v, sem):
    wid = lax.axis_index("s") * NC + lax.axis_index("c")
    pltpu.sync_copy(pos_hbm, pos_v)
    pltpu.sync_copy(wmap_hbm, wmap_v)

    def chunk(k, carry):
        b0 = wid * BPW + k * NB
        pltpu.sync_copy(inp_hbm.at[pl.ds(b0 * CA, NB * CA)], inp_v)

        for g in range(EV_PAD // 16):
            w0 = wmap_v[pl.ds(g * 16, 16)]
            f0 = plsc.load_gather(inp_v, [w0])
            f1 = plsc.load_gather(inp_v, [w0 + 1])
            r, off = g // 7, (g % 7) * 16
            idx0_v[r, pl.ds(off, 16)] = f0.astype(jnp.int32)
            idx1_v[r, pl.ds(off, 16)] = f1.astype(jnp.int32)

        cps = []
        for j in range(2):
            cps.append(pltpu.async_copy(
                ta_hbm.at[idx0_v.at[j]], st0_v.at[pl.ds(j * HALF, HALF)], sem))
            cps.append(pltpu.async_copy(
                tr_hbm.at[idx1_v.at[j]], st1_v.at[pl.ds(j * HALF, HALF)], sem))
        for cp in cps:
            cp.wait()

        def ev_body(c, carry2):
            for b in range(NB):
                e = b * C + c
                wn = jnp.full((16,), b * CA + 2, jnp.int32) + c * A
                n0 = plsc.load_gather(inp_v, [wn])
                n1 = plsc.load_gather(inp_v, [wn + 1])
                for j in range(D // 16):
                    p = pos_v[c, pl.ds(j * 16, 16)]
                    v0 = st0_v[e, pl.ds(j * 16, 16)]
                    v1 = st1_v[e, pl.ds(j * 16, 16)]
                    buf_v[b, c, 0, pl.ds(j * 16, 16)] = v0 + p
                    buf_v[b, c, 1, pl.ds(j * 16, 16)] = v1 + p
                    buf_v[b, c, 2, pl.ds(j * 16, 16)] = n0 + p
                    buf_v[b, c, 3, pl.ds(j * 16, 16)] = n1 + p
            return carry2

        lax.fori_loop(0, C, ev_body, 0)

        pltpu.sync_copy(buf_v, out_hbm.at[pl.ds(b0, NB)])
        return carry

    lax.fori_loop(0, NCHUNK, chunk, 0)


def kernel(inputs, table_activity, table_resource):
    pos = jnp.asarray(_POS)
    wmap = jnp.asarray(_WMAP)
    mesh = plsc.VectorSubcoreMesh(core_axis_name="c", subcore_axis_name="s")
    k = functools.partial(
        pl.kernel,
        out_type=jax.ShapeDtypeStruct((B, C, A, D), jnp.float32),
        mesh=mesh,
        compiler_params=pltpu.CompilerParams(use_tc_tiling_on_sc=False,
                                             needs_layout_passes=False),
        scratch_types=[
            pltpu.VMEM((NB * CA,), jnp.float32),
            pltpu.VMEM((2, HALF), jnp.int32),
            pltpu.VMEM((2, HALF), jnp.int32),
            pltpu.VMEM((EV_PAD, D), jnp.float32),
            pltpu.VMEM((EV_PAD, D), jnp.float32),
            pltpu.VMEM((NB, C, A, D), jnp.float32),
            pltpu.VMEM((C, D), jnp.float32),
            pltpu.VMEM((EV_PAD,), jnp.int32),
            pltpu.SemaphoreType.DMA,
        ],
    )(_sc_body)
    out = k(inputs.reshape(B * CA), table_activity, table_resource, pos, wmap)
    return out.reshape(B, CA, D)
